# baseline (device time: 48359 ns/iter reference)
import jax
import jax.numpy as jnp
from jax import lax
from jax.experimental import pallas as pl
from jax.experimental.pallas import tpu as pltpu

N_DEV = 32
B, SQ, SKV, DH = 2, 128, 128, 64
H_LOC = 4
ROWS = B * SQ
CHUNK = ROWS // N_DEV
D_MODEL = 512


def kernel(x, Wq, K_ext, V_ext, Wo):
    idx = lax.axis_index("i")
    K_loc = jnp.transpose(
        lax.dynamic_slice_in_dim(K_ext, idx * H_LOC, H_LOC, axis=2), (0, 2, 1, 3)
    )
    V_loc = jnp.transpose(
        lax.dynamic_slice_in_dim(V_ext, idx * H_LOC, H_LOC, axis=2), (0, 2, 1, 3)
    )
    xf = x.reshape(ROWS, D_MODEL)

    def body(x_ref, wq_ref, k_ref, v_ref, wo_ref, out_ref,
             partial_ref, acc_ref, gath_ref, send_sems, recv1, recv2):
        my = lax.axis_index("i")

        Q = jnp.dot(x_ref[...], wq_ref[...], preferred_element_type=jnp.float32)
        brows = []
        for b in range(B):
            cols = []
            for h in range(H_LOC):
                q = Q[b * SQ:(b + 1) * SQ, h * DH:(h + 1) * DH]
                k = k_ref[b, h, :, :]
                s = lax.dot_general(
                    q, k, (((1,), (1,)), ((), ())),
                    preferred_element_type=jnp.float32,
                ) * 0.125
                m = jnp.max(s, axis=1, keepdims=True)
                w = jnp.exp(s - m)
                w = w / jnp.sum(w, axis=1, keepdims=True)
                cols.append(jnp.dot(w, v_ref[b, h, :, :],
                                    preferred_element_type=jnp.float32))
            brows.append(jnp.concatenate(cols, axis=1))
        ctx = jnp.concatenate(brows, axis=0)
        partial_ref[...] = jnp.dot(ctx, wo_ref[...],
                                   preferred_element_type=jnp.float32)

        p1 = []
        for kk in range(1, N_DEV):
            j = lax.rem(my + kk, N_DEV)
            rdma = pltpu.make_async_remote_copy(
                src_ref=partial_ref.at[pl.ds(j * CHUNK, CHUNK), :],
                dst_ref=acc_ref.at[my],
                send_sem=send_sems.at[kk - 1],
                recv_sem=recv1.at[my],
                device_id=(j,),
                device_id_type=pl.DeviceIdType.MESH,
            )
            rdma.start()
            p1.append(rdma)
        acc_ref[my, :, :] = partial_ref[pl.ds(my * CHUNK, CHUNK), :]

        for kk in range(1, N_DEV):
            s = lax.rem(my - kk + N_DEV, N_DEV)
            pltpu.make_async_remote_copy(
                src_ref=acc_ref.at[s],
                dst_ref=acc_ref.at[s],
                send_sem=send_sems.at[0],
                recv_sem=recv1.at[s],
                device_id=(my,),
                device_id_type=pl.DeviceIdType.MESH,
            ).wait_recv()

        for r in p1:
            r.wait_send()

        red = jnp.sum(acc_ref[...], axis=0)
        gath_ref[pl.ds(my * CHUNK, CHUNK), :] = red

        p2 = []
        for kk in range(1, N_DEV):
            j = lax.rem(my + kk, N_DEV)
            rdma = pltpu.make_async_remote_copy(
                src_ref=gath_ref.at[pl.ds(my * CHUNK, CHUNK), :],
                dst_ref=gath_ref.at[pl.ds(my * CHUNK, CHUNK), :],
                send_sem=send_sems.at[kk - 1],
                recv_sem=recv2.at[my],
                device_id=(j,),
                device_id_type=pl.DeviceIdType.MESH,
            )
            rdma.start()
            p2.append(rdma)
        for kk in range(1, N_DEV):
            s = lax.rem(my - kk + N_DEV, N_DEV)
            pltpu.make_async_remote_copy(
                src_ref=gath_ref.at[pl.ds(s * CHUNK, CHUNK), :],
                dst_ref=gath_ref.at[pl.ds(s * CHUNK, CHUNK), :],
                send_sem=send_sems.at[0],
                recv_sem=recv2.at[s],
                device_id=(my,),
                device_id_type=pl.DeviceIdType.MESH,
            ).wait_recv()

        out_ref[...] = gath_ref[...]
        for r in p2:
            r.wait_send()

    out = pl.pallas_call(
        body,
        out_shape=jax.ShapeDtypeStruct((ROWS, D_MODEL), jnp.float32),
        in_specs=[pl.BlockSpec(memory_space=pltpu.VMEM)] * 5,
        out_specs=pl.BlockSpec(memory_space=pltpu.VMEM),
        scratch_shapes=[
            pltpu.VMEM((ROWS, D_MODEL), jnp.float32),
            pltpu.VMEM((N_DEV, CHUNK, D_MODEL), jnp.float32),
            pltpu.VMEM((ROWS, D_MODEL), jnp.float32),
            pltpu.SemaphoreType.DMA((N_DEV - 1,)),
            pltpu.SemaphoreType.DMA((N_DEV,)),
            pltpu.SemaphoreType.DMA((N_DEV,)),
        ],
    )(xf, Wq, K_loc, V_loc, Wo)
    return out.reshape(B, SQ, D_MODEL)


# device time: 48327 ns/iter; 1.0007x vs baseline; 1.0007x over previous
import jax
import jax.numpy as jnp
from jax import lax
from jax.experimental import pallas as pl
from jax.experimental.pallas import tpu as pltpu

N_DEV = 32
B, SQ, SKV, DH = 2, 128, 128, 64
H_LOC = 4
ROWS = B * SQ
CHUNK = ROWS // N_DEV
D_MODEL = 512


def kernel(x, Wq, K_ext, V_ext, Wo):
    idx = lax.axis_index("i")
    K_loc = lax.dynamic_slice_in_dim(K_ext, idx * H_LOC, H_LOC, axis=2)
    V_loc = lax.dynamic_slice_in_dim(V_ext, idx * H_LOC, H_LOC, axis=2)
    xf = x.reshape(ROWS, D_MODEL)

    def body(x_ref, wq_ref, k_ref, v_ref, wo_ref, out_ref,
             partial_ref, acc_ref, gath_ref, send_sems, recv1, recv2):
        my = lax.axis_index("i")

        Q = jnp.dot(x_ref[...], wq_ref[...], preferred_element_type=jnp.float32)
        brows = []
        for b in range(B):
            cols = []
            for h in range(H_LOC):
                q = Q[b * SQ:(b + 1) * SQ, h * DH:(h + 1) * DH]
                k = k_ref[b, :, h, :]
                s = lax.dot_general(
                    q, k, (((1,), (1,)), ((), ())),
                    preferred_element_type=jnp.float32,
                ) * 0.125
                m = jnp.max(s, axis=1, keepdims=True)
                w = jnp.exp(s - m)
                w = w / jnp.sum(w, axis=1, keepdims=True)
                cols.append(jnp.dot(w, v_ref[b, :, h, :],
                                    preferred_element_type=jnp.float32))
            brows.append(jnp.concatenate(cols, axis=1))
        ctx = jnp.concatenate(brows, axis=0)
        partial_ref[...] = jnp.dot(ctx, wo_ref[...],
                                   preferred_element_type=jnp.float32)

        p1 = []
        for kk in range(1, N_DEV):
            j = lax.rem(my + kk, N_DEV)
            rdma = pltpu.make_async_remote_copy(
                src_ref=partial_ref.at[pl.ds(j * CHUNK, CHUNK), :],
                dst_ref=acc_ref.at[my],
                send_sem=send_sems.at[kk - 1],
                recv_sem=recv1.at[my],
                device_id=(j,),
                device_id_type=pl.DeviceIdType.MESH,
            )
            rdma.start()
            p1.append(rdma)
        acc_ref[my, :, :] = partial_ref[pl.ds(my * CHUNK, CHUNK), :]

        for kk in range(1, N_DEV):
            s = lax.rem(my - kk + N_DEV, N_DEV)
            pltpu.make_async_remote_copy(
                src_ref=acc_ref.at[s],
                dst_ref=acc_ref.at[s],
                send_sem=send_sems.at[0],
                recv_sem=recv1.at[s],
                device_id=(my,),
                device_id_type=pl.DeviceIdType.MESH,
            ).wait_recv()

        for r in p1:
            r.wait_send()

        red = jnp.sum(acc_ref[...], axis=0)
        gath_ref[pl.ds(my * CHUNK, CHUNK), :] = red

        p2 = []
        for kk in range(1, N_DEV):
            j = lax.rem(my + kk, N_DEV)
            rdma = pltpu.make_async_remote_copy(
                src_ref=gath_ref.at[pl.ds(my * CHUNK, CHUNK), :],
                dst_ref=gath_ref.at[pl.ds(my * CHUNK, CHUNK), :],
                send_sem=send_sems.at[kk - 1],
                recv_sem=recv2.at[my],
                device_id=(j,),
                device_id_type=pl.DeviceIdType.MESH,
            )
            rdma.start()
            p2.append(rdma)
        for kk in range(1, N_DEV):
            s = lax.rem(my - kk + N_DEV, N_DEV)
            pltpu.make_async_remote_copy(
                src_ref=gath_ref.at[pl.ds(s * CHUNK, CHUNK), :],
                dst_ref=gath_ref.at[pl.ds(s * CHUNK, CHUNK), :],
                send_sem=send_sems.at[0],
                recv_sem=recv2.at[s],
                device_id=(my,),
                device_id_type=pl.DeviceIdType.MESH,
            ).wait_recv()

        out_ref[...] = gath_ref[...]
        for r in p2:
            r.wait_send()

    out = pl.pallas_call(
        body,
        out_shape=jax.ShapeDtypeStruct((ROWS, D_MODEL), jnp.float32),
        in_specs=[pl.BlockSpec(memory_space=pltpu.VMEM)] * 5,
        out_specs=pl.BlockSpec(memory_space=pltpu.VMEM),
        scratch_shapes=[
            pltpu.VMEM((ROWS, D_MODEL), jnp.float32),
            pltpu.VMEM((N_DEV, CHUNK, D_MODEL), jnp.float32),
            pltpu.VMEM((ROWS, D_MODEL), jnp.float32),
            pltpu.SemaphoreType.DMA((N_DEV - 1,)),
            pltpu.SemaphoreType.DMA((N_DEV,)),
            pltpu.SemaphoreType.DMA((N_DEV,)),
        ],
    )(xf, Wq, K_loc, V_loc, Wo)
    return out.reshape(B, SQ, D_MODEL)
